# baseline (speedup 1.0000x reference)
import functools

import jax
import jax.numpy as jnp
from jax import lax
from jax.experimental import pallas as pl
from jax.experimental.pallas import tpu as pltpu
from jax.experimental.pallas import tpu_sc as plsc

_B = 16384
_NC = 2
_NS = 16
_NW = _NC * _NS
_BPW = _B // _NW
_EPS = 1e-8


def _gather_body(ui_flat, ii_flat, U0, U1, U2, I0, I1, I2,
                 o_u0, o_u1, o_u2, o_i0, o_i1, o_i2,
                 uraw, iraw,
                 idx0, idx1, idx2, idx3, idx4, idx5,
                 bu0, bu1, bu2, bi0, bi1, bi2, sem):
    wid = lax.axis_index("s") * _NC + lax.axis_index("c")
    base = wid * _BPW
    pltpu.sync_copy(ui_flat.at[pl.ds(base * 3, _BPW * 3)], uraw)
    pltpu.sync_copy(ii_flat.at[pl.ds(base * 3, _BPW * 3)], iraw)
    lane = lax.iota(jnp.int32, 16)
    for raw, cols in ((uraw, (idx0, idx1, idx2)), (iraw, (idx3, idx4, idx5))):
        for c, idxv in enumerate(cols):
            for j in range(0, _BPW, 16):
                pos = (j + lane) * 3 + c
                idxv[pl.ds(j, 16)] = plsc.load_gather(raw, [pos])
    jobs = (
        (U0, o_u0, idx0, bu0),
        (U1, o_u1, idx1, bu1),
        (U2, o_u2, idx2, bu2),
        (I0, o_i0, idx3, bi0),
        (I1, o_i1, idx4, bi1),
        (I2, o_i2, idx5, bi2),
    )
    copies = []
    for tab, _, idx_v, buf in jobs:
        copies.append(pltpu.async_copy(tab.at[idx_v], buf, sem))
    for (tab, out, idx_v, buf), cp in zip(jobs, copies):
        cp.wait()
        pltpu.sync_copy(buf, out.at[pl.ds(base, _BPW)])


@functools.cache
def _make_gather():
    return functools.partial(
        pl.kernel,
        out_type=[
            jax.ShapeDtypeStruct((_B, 32), jnp.float32),
            jax.ShapeDtypeStruct((_B, 16), jnp.float32),
            jax.ShapeDtypeStruct((_B, 16), jnp.float32),
            jax.ShapeDtypeStruct((_B, 32), jnp.float32),
            jax.ShapeDtypeStruct((_B, 16), jnp.float32),
            jax.ShapeDtypeStruct((_B, 16), jnp.float32),
        ],
        mesh=plsc.VectorSubcoreMesh(core_axis_name="c", subcore_axis_name="s"),
        compiler_params=pltpu.CompilerParams(
            use_tc_tiling_on_sc=False, needs_layout_passes=False),
        scratch_types=[
            pltpu.VMEM((_BPW * 3,), jnp.int32),
            pltpu.VMEM((_BPW * 3,), jnp.int32),
            pltpu.VMEM((_BPW,), jnp.int32),
            pltpu.VMEM((_BPW,), jnp.int32),
            pltpu.VMEM((_BPW,), jnp.int32),
            pltpu.VMEM((_BPW,), jnp.int32),
            pltpu.VMEM((_BPW,), jnp.int32),
            pltpu.VMEM((_BPW,), jnp.int32),
            pltpu.VMEM((_BPW, 32), jnp.float32),
            pltpu.VMEM((_BPW, 16), jnp.float32),
            pltpu.VMEM((_BPW, 16), jnp.float32),
            pltpu.VMEM((_BPW, 32), jnp.float32),
            pltpu.VMEM((_BPW, 16), jnp.float32),
            pltpu.VMEM((_BPW, 16), jnp.float32),
            pltpu.SemaphoreType.DMA,
        ],
    )(_gather_body)


def _mlp_body(u0, u1, u2, i0, i1, i2, W1, b1, W2, b2, W3, b3, Wo, bo, out):
    x = jnp.concatenate(
        [u0[...], u1[...], u2[...], i0[...], i1[...], i2[...]], axis=1)
    a = x[:, 33:64]
    s = jnp.sum(a * a, axis=1, keepdims=True)
    na = jnp.sqrt(s)
    d = jnp.maximum(na, _EPS)
    cos = s / (d * d)
    h = jnp.maximum(jnp.dot(x, W1[...], preferred_element_type=jnp.float32) + b1[...], 0.0)
    h = jnp.maximum(jnp.dot(h, W2[...], preferred_element_type=jnp.float32) + b2[...], 0.0)
    h = jnp.maximum(jnp.dot(h, W3[...], preferred_element_type=jnp.float32) + b3[...], 0.0)
    hc = jnp.concatenate([h, cos], axis=1)
    out[...] = jnp.dot(hc, Wo[...], preferred_element_type=jnp.float32) + bo[...]


def _mlp(ue0, ue1, ue2, ie0, ie1, ie2, W1, b1, W2, b2, W3, b3, Wo, bo):
    blk = 2048
    grid = (_B // blk,)
    row = lambda w: pl.BlockSpec((blk, w), lambda i: (i, 0))
    rep = lambda a, b: pl.BlockSpec((a, b), lambda i: (0, 0))
    return pl.pallas_call(
        _mlp_body,
        grid=grid,
        in_specs=[
            row(32), row(16), row(16), row(32), row(16), row(16),
            rep(128, 64), rep(1, 64), rep(64, 32), rep(1, 32),
            rep(32, 16), rep(1, 16), rep(17, 1), rep(1, 1),
        ],
        out_specs=pl.BlockSpec((blk, 1), lambda i: (i, 0)),
        out_shape=jax.ShapeDtypeStruct((_B, 1), jnp.float32),
    )(ue0, ue1, ue2, ie0, ie1, ie2, W1, b1, W2, b2, W3, b3, Wo, bo)


def kernel(user_idx, item_idx, U0, U1, U2, I0, I1, I2,
           W1, b1, W2, b2, W3, b3, Wo, bo):
    ui = user_idx.astype(jnp.int32).reshape(-1)
    ii = item_idx.astype(jnp.int32).reshape(-1)
    ue0, ue1, ue2, ie0, ie1, ie2 = _make_gather()(
        ui, ii, U0, U1, U2, I0, I1, I2)
    return _mlp(ue0, ue1, ue2, ie0, ie1, ie2,
                W1, b1.reshape(1, -1), W2, b2.reshape(1, -1),
                W3, b3.reshape(1, -1), Wo, bo.reshape(1, 1))


# speedup vs baseline: 1.0655x; 1.0655x over previous
import functools

import jax
import jax.numpy as jnp
from jax import lax
from jax.experimental import pallas as pl
from jax.experimental.pallas import tpu as pltpu
from jax.experimental.pallas import tpu_sc as plsc

_B = 16384
_NC = 2
_NS = 16
_NW = _NC * _NS
_BPW = _B // _NW
_EPS = 1e-8


def _gather_body(u0i, u1i, u2i, i0i, i1i, i2i, U0, U1, U2, I0, I1, I2,
                 o_u0, o_u1, o_u2, o_i0, o_i1, o_i2,
                 idx0, idx1, idx2, idx3, idx4, idx5,
                 bu0, bu1, bu2, bi0, bi1, bi2, sem, semw):
    wid = lax.axis_index("s") * _NC + lax.axis_index("c")
    base = wid * _BPW
    jobs = (
        (u0i, U0, o_u0, idx0, bu0),
        (i0i, I0, o_i0, idx3, bi0),
        (u1i, U1, o_u1, idx1, bu1),
        (u2i, U2, o_u2, idx2, bu2),
        (i1i, I1, o_i1, idx4, bi1),
        (i2i, I2, o_i2, idx5, bi2),
    )
    copies = []
    for idx_hbm, tab, _, idx_v, buf in jobs:
        pltpu.sync_copy(idx_hbm.at[pl.ds(base, _BPW)], idx_v)
        copies.append(pltpu.async_copy(tab.at[idx_v], buf, sem))
    wbs = []
    for (idx_hbm, tab, out, idx_v, buf), cp in zip(jobs, copies):
        cp.wait()
        wbs.append(pltpu.async_copy(buf, out.at[pl.ds(base, _BPW)], semw))
    for wb in wbs:
        wb.wait()


@functools.cache
def _make_gather():
    return functools.partial(
        pl.kernel,
        out_type=[
            jax.ShapeDtypeStruct((_B, 32), jnp.float32),
            jax.ShapeDtypeStruct((_B, 16), jnp.float32),
            jax.ShapeDtypeStruct((_B, 16), jnp.float32),
            jax.ShapeDtypeStruct((_B, 32), jnp.float32),
            jax.ShapeDtypeStruct((_B, 16), jnp.float32),
            jax.ShapeDtypeStruct((_B, 16), jnp.float32),
        ],
        mesh=plsc.VectorSubcoreMesh(core_axis_name="c", subcore_axis_name="s"),
        compiler_params=pltpu.CompilerParams(use_tc_tiling_on_sc=False),
        scratch_types=[
            pltpu.VMEM((_BPW,), jnp.int32),
            pltpu.VMEM((_BPW,), jnp.int32),
            pltpu.VMEM((_BPW,), jnp.int32),
            pltpu.VMEM((_BPW,), jnp.int32),
            pltpu.VMEM((_BPW,), jnp.int32),
            pltpu.VMEM((_BPW,), jnp.int32),
            pltpu.VMEM((_BPW, 32), jnp.float32),
            pltpu.VMEM((_BPW, 16), jnp.float32),
            pltpu.VMEM((_BPW, 16), jnp.float32),
            pltpu.VMEM((_BPW, 32), jnp.float32),
            pltpu.VMEM((_BPW, 16), jnp.float32),
            pltpu.VMEM((_BPW, 16), jnp.float32),
            pltpu.SemaphoreType.DMA,
            pltpu.SemaphoreType.DMA,
        ],
    )(_gather_body)


def _mlp_body(u0, u1, u2, i0, i1, i2, W1, b1, W2, b2, W3, b3, Wo, bo, out):
    x = jnp.concatenate(
        [u0[...], u1[...], u2[...], i0[...], i1[...], i2[...]], axis=1)
    a = x[:, 33:64]
    s = jnp.sum(a * a, axis=1, keepdims=True)
    na = jnp.sqrt(s)
    d = jnp.maximum(na, _EPS)
    cos = s / (d * d)
    h = jnp.maximum(jnp.dot(x, W1[...], preferred_element_type=jnp.float32) + b1[...], 0.0)
    h = jnp.maximum(jnp.dot(h, W2[...], preferred_element_type=jnp.float32) + b2[...], 0.0)
    h = jnp.maximum(jnp.dot(h, W3[...], preferred_element_type=jnp.float32) + b3[...], 0.0)
    hc = jnp.concatenate([h, cos], axis=1)
    out[...] = jnp.dot(hc, Wo[...], preferred_element_type=jnp.float32) + bo[...]


def _mlp(ue0, ue1, ue2, ie0, ie1, ie2, W1, b1, W2, b2, W3, b3, Wo, bo):
    blk = 2048
    grid = (_B // blk,)
    row = lambda w: pl.BlockSpec((blk, w), lambda i: (i, 0))
    rep = lambda a, b: pl.BlockSpec((a, b), lambda i: (0, 0))
    return pl.pallas_call(
        _mlp_body,
        grid=grid,
        in_specs=[
            row(32), row(16), row(16), row(32), row(16), row(16),
            rep(128, 64), rep(1, 64), rep(64, 32), rep(1, 32),
            rep(32, 16), rep(1, 16), rep(17, 1), rep(1, 1),
        ],
        out_specs=pl.BlockSpec((blk, 1), lambda i: (i, 0)),
        out_shape=jax.ShapeDtypeStruct((_B, 1), jnp.float32),
    )(ue0, ue1, ue2, ie0, ie1, ie2, W1, b1, W2, b2, W3, b3, Wo, bo)


def kernel(user_idx, item_idx, U0, U1, U2, I0, I1, I2,
           W1, b1, W2, b2, W3, b3, Wo, bo):
    ui = user_idx.astype(jnp.int32)
    ii = item_idx.astype(jnp.int32)
    ue0, ue1, ue2, ie0, ie1, ie2 = _make_gather()(
        ui[:, 0], ui[:, 1], ui[:, 2], ii[:, 0], ii[:, 1], ii[:, 2],
        U0, U1, U2, I0, I1, I2)
    return _mlp(ue0, ue1, ue2, ie0, ie1, ie2,
                W1, b1.reshape(1, -1), W2, b2.reshape(1, -1),
                W3, b3.reshape(1, -1), Wo, bo.reshape(1, 1))


# speedup vs baseline: 2.0981x; 1.9692x over previous
import functools

import jax
import jax.numpy as jnp
from jax import lax
from jax.experimental import pallas as pl
from jax.experimental.pallas import tpu as pltpu
from jax.experimental.pallas import tpu_sc as plsc

_B = 16384
_V = 100000
_NC = 2
_NS = 16
_NW = _NC * _NS
_HALF = _B // 2
_EPS = 1e-8


def _gather_body(u0i, u1i, u2i, i0i, i1i, i2i,
                 U0t, U1t, U2t, I0t, I1t, I2t,
                 out, trow, idxv, outv, osem):
    wid = lax.axis_index("s") * _NC + lax.axis_index("c")
    sub = wid - 16

    def stage(tab, row, idx_hbm):
        pltpu.sync_copy(tab.at[row], trow)
        pltpu.sync_copy(idx_hbm, idxv)

    def gather_to(drow):
        for half in range(2):
            @pl.loop(0, _HALF // 16)
            def _(i):
                v = idxv[pl.ds(half * _HALF + i * 16, 16)]
                outv[pl.ds(i * 16, 16)] = plsc.load_gather(trow, [v])
            pltpu.sync_copy(outv, out.at[drow, half])

    stage(U0t, wid, u0i)
    gather_to(wid)
    @pl.when(wid < 16)
    def _():
        stage(U1t, wid, u1i)
    @pl.when(wid >= 16)
    def _():
        stage(U2t, sub, u2i)
    gather_to(32 + wid)
    stage(I0t, wid, i0i)
    gather_to(64 + wid)
    @pl.when(wid < 16)
    def _():
        stage(I1t, wid, i1i)
    @pl.when(wid >= 16)
    def _():
        stage(I2t, sub, i2i)
    gather_to(96 + wid)
    del osem


@functools.cache
def _make_gather():
    return functools.partial(
        pl.kernel,
        out_type=jax.ShapeDtypeStruct((128, 2, _HALF), jnp.float32),
        mesh=plsc.VectorSubcoreMesh(core_axis_name="c", subcore_axis_name="s"),
        compiler_params=pltpu.CompilerParams(
            use_tc_tiling_on_sc=False, needs_layout_passes=False),
        scratch_types=[
            pltpu.VMEM((_V,), jnp.float32),
            pltpu.VMEM((_B,), jnp.int32),
            pltpu.VMEM((_HALF,), jnp.float32),
            pltpu.SemaphoreType.DMA,
        ],
    )(_gather_body)


def _mlp_body(xT, W1t, b1, W2t, b2, W3t, b3, Wot, bo, out):
    x = xT[...]
    a = x[33:64, :]
    s = jnp.sum(a * a, axis=0, keepdims=True)
    na = jnp.sqrt(s)
    d = jnp.maximum(na, _EPS)
    cos = s / (d * d)
    h = jnp.maximum(jnp.dot(W1t[...], x, preferred_element_type=jnp.float32) + b1[...], 0.0)
    h = jnp.maximum(jnp.dot(W2t[...], h, preferred_element_type=jnp.float32) + b2[...], 0.0)
    h = jnp.maximum(jnp.dot(W3t[...], h, preferred_element_type=jnp.float32) + b3[...], 0.0)
    hc = jnp.concatenate([h, cos], axis=0)
    out[...] = jnp.dot(Wot[...], hc, preferred_element_type=jnp.float32) + bo[...]


def _mlp_t(xT, W1t, b1, W2t, b2, W3t, b3, Wot, bo):
    blk = 4096
    grid = (_B // blk,)
    col = lambda h: pl.BlockSpec((h, blk), lambda i: (0, i))
    rep = lambda a, b: pl.BlockSpec((a, b), lambda i: (0, 0))
    return pl.pallas_call(
        _mlp_body,
        grid=grid,
        in_specs=[
            col(128),
            rep(64, 128), rep(64, 1), rep(32, 64), rep(32, 1),
            rep(16, 32), rep(16, 1), rep(1, 17), rep(1, 1),
        ],
        out_specs=pl.BlockSpec((1, blk), lambda i: (0, i)),
        out_shape=jax.ShapeDtypeStruct((1, _B), jnp.float32),
    )(xT, W1t, b1, W2t, b2, W3t, b3, Wot, bo)


def kernel(user_idx, item_idx, U0, U1, U2, I0, I1, I2,
           W1, b1, W2, b2, W3, b3, Wo, bo):
    ui = user_idx.astype(jnp.int32)
    ii = item_idx.astype(jnp.int32)
    xT3 = _make_gather()(
        ui[:, 0], ui[:, 1], ui[:, 2], ii[:, 0], ii[:, 1], ii[:, 2],
        U0.T, U1.T, U2.T, I0.T, I1.T, I2.T)
    xT = xT3.reshape(128, _B)
    outT = _mlp_t(xT, W1.T, b1.reshape(-1, 1), W2.T, b2.reshape(-1, 1),
                  W3.T, b3.reshape(-1, 1), Wo.T, bo.reshape(1, 1))
    return outT.reshape(_B, 1)


# speedup vs baseline: 3.4428x; 1.6409x over previous
import functools

import jax
import jax.numpy as jnp
from jax import lax
from jax.experimental import pallas as pl
from jax.experimental.pallas import tpu as pltpu
from jax.experimental.pallas import tpu_sc as plsc

_B = 16384
_V = 100000
_NC = 2
_NS = 16
_NW = _NC * _NS
_HALF = _B // 2
_EPS = 1e-8


def _gather_body(u0i, u1i, u2i, i0i, i1i, i2i,
                 U0t, U1t, U2t, I0t, I1t, I2t,
                 out, trow, idxv, outv, osem):
    wid = lax.axis_index("s") * _NC + lax.axis_index("c")
    sub = wid - 16

    def stage(tab, row, idx_hbm):
        pltpu.sync_copy(tab.at[row], trow)
        pltpu.sync_copy(idx_hbm, idxv)

    def gather_to(drow):
        for half in range(2):
            @pl.loop(0, _HALF // 16)
            def _(i):
                v = idxv[pl.ds(half * _HALF + i * 16, 16)]
                outv[pl.ds(i * 16, 16)] = plsc.load_gather(trow, [v])
            pltpu.sync_copy(outv, out.at[drow, half])

    stage(U0t, wid, u0i)
    gather_to(wid)
    @pl.when(wid < 16)
    def _():
        stage(U1t, wid, u1i)
    @pl.when(wid >= 16)
    def _():
        stage(U2t, sub, u2i)
    gather_to(32 + wid)
    stage(I0t, wid, i0i)
    gather_to(64 + wid)
    @pl.when(wid < 16)
    def _():
        stage(I1t, wid, i1i)
    @pl.when(wid >= 16)
    def _():
        stage(I2t, sub, i2i)
    gather_to(96 + wid)
    del osem


@functools.cache
def _make_gather():
    return functools.partial(
        pl.kernel,
        out_type=jax.ShapeDtypeStruct((128, 2, _HALF), jnp.float32),
        mesh=plsc.VectorSubcoreMesh(core_axis_name="c", subcore_axis_name="s"),
        compiler_params=pltpu.CompilerParams(
            use_tc_tiling_on_sc=True, needs_layout_passes=False),
        scratch_types=[
            pltpu.VMEM((_V,), jnp.float32),
            pltpu.VMEM((_B,), jnp.int32),
            pltpu.VMEM((_HALF,), jnp.float32),
            pltpu.SemaphoreType.DMA,
        ],
    )(_gather_body)


def _mlp_body(xT, W1t, b1, W2t, b2, W3t, b3, Wot, bo, out):
    x = xT[...]
    a = x[33:64, :]
    s = jnp.sum(a * a, axis=0, keepdims=True)
    na = jnp.sqrt(s)
    d = jnp.maximum(na, _EPS)
    cos = s / (d * d)
    h = jnp.maximum(jnp.dot(W1t[...], x, preferred_element_type=jnp.float32) + b1[...], 0.0)
    h = jnp.maximum(jnp.dot(W2t[...], h, preferred_element_type=jnp.float32) + b2[...], 0.0)
    h = jnp.maximum(jnp.dot(W3t[...], h, preferred_element_type=jnp.float32) + b3[...], 0.0)
    hc = jnp.concatenate([h, cos], axis=0)
    out[...] = jnp.dot(Wot[...], hc, preferred_element_type=jnp.float32) + bo[...]


def _mlp_t(xT, W1t, b1, W2t, b2, W3t, b3, Wot, bo):
    blk = 4096
    grid = (_B // blk,)
    col = lambda h: pl.BlockSpec((h, blk), lambda i: (0, i))
    rep = lambda a, b: pl.BlockSpec((a, b), lambda i: (0, 0))
    return pl.pallas_call(
        _mlp_body,
        grid=grid,
        in_specs=[
            col(128),
            rep(64, 128), rep(64, 1), rep(32, 64), rep(32, 1),
            rep(16, 32), rep(16, 1), rep(1, 17), rep(1, 1),
        ],
        out_specs=pl.BlockSpec((1, blk), lambda i: (0, i)),
        out_shape=jax.ShapeDtypeStruct((1, _B), jnp.float32),
    )(xT, W1t, b1, W2t, b2, W3t, b3, Wot, bo)


def kernel(user_idx, item_idx, U0, U1, U2, I0, I1, I2,
           W1, b1, W2, b2, W3, b3, Wo, bo):
    ui = user_idx.astype(jnp.int32)
    ii = item_idx.astype(jnp.int32)
    xT3 = _make_gather()(
        ui[:, 0], ui[:, 1], ui[:, 2], ii[:, 0], ii[:, 1], ii[:, 2],
        U0.T, U1.T, U2.T, I0.T, I1.T, I2.T)
    xT = xT3.reshape(128, _B)
    outT = _mlp_t(xT, W1.T, b1.reshape(-1, 1), W2.T, b2.reshape(-1, 1),
                  W3.T, b3.reshape(-1, 1), Wo.T, bo.reshape(1, 1))
    return outT.reshape(_B, 1)


# speedup vs baseline: 3.4673x; 1.0071x over previous
import functools

import jax
import jax.numpy as jnp
from jax import lax
from jax.experimental import pallas as pl
from jax.experimental.pallas import tpu as pltpu
from jax.experimental.pallas import tpu_sc as plsc

_B = 16384
_V = 100000
_NC = 2
_NS = 16
_NW = _NC * _NS
_HALF = _B // 2
_EPS = 1e-8


def _gather_body(u0i, u1i, u2i, i0i, i1i, i2i,
                 U0t, U1t, U2t, I0t, I1t, I2t,
                 out, trow, idxv, outv, osem):
    wid = lax.axis_index("s") * _NC + lax.axis_index("c")
    sub = wid - 16

    def stage(tab, row, idx_hbm):
        pltpu.sync_copy(tab.at[row], trow)
        pltpu.sync_copy(idx_hbm, idxv)

    def gather_to(drow):
        for half in range(2):
            @pl.loop(0, _HALF // 16)
            def _(i):
                v = idxv[pl.ds(half * _HALF + i * 16, 16)]
                outv[pl.ds(i * 16, 16)] = plsc.load_gather(trow, [v])
            pltpu.sync_copy(outv, out.at[drow, half])

    stage(U0t, wid, u0i)
    gather_to(wid)
    @pl.when(wid < 16)
    def _():
        stage(U1t, wid, u1i)
    @pl.when(wid >= 16)
    def _():
        stage(U2t, sub, u2i)
    gather_to(32 + wid)
    stage(I0t, wid, i0i)
    gather_to(64 + wid)
    @pl.when(wid < 16)
    def _():
        stage(I1t, wid, i1i)
    @pl.when(wid >= 16)
    def _():
        stage(I2t, sub, i2i)
    gather_to(96 + wid)
    del osem


@functools.cache
def _make_gather():
    return functools.partial(
        pl.kernel,
        out_type=jax.ShapeDtypeStruct((128, 2, _HALF), jnp.float32),
        mesh=plsc.VectorSubcoreMesh(core_axis_name="c", subcore_axis_name="s"),
        compiler_params=pltpu.CompilerParams(
            use_tc_tiling_on_sc=True, needs_layout_passes=False),
        scratch_types=[
            pltpu.VMEM((_V,), jnp.float32),
            pltpu.VMEM((_B,), jnp.int32),
            pltpu.VMEM((_HALF,), jnp.float32),
            pltpu.SemaphoreType.DMA,
        ],
    )(_gather_body)


def _mlp_body(xT, W1t, b1, W2t, b2, W3t, b3, Wot, bo, out):
    x = xT[...]
    a = x[33:64, :]
    s = jnp.sum(a * a, axis=0, keepdims=True)
    na = jnp.sqrt(s)
    d = jnp.maximum(na, _EPS)
    cos = s / (d * d)
    h = jnp.maximum(jnp.dot(W1t[...], x, preferred_element_type=jnp.float32) + b1[...], 0.0)
    h = jnp.maximum(jnp.dot(W2t[...], h, preferred_element_type=jnp.float32) + b2[...], 0.0)
    h = jnp.maximum(jnp.dot(W3t[...], h, preferred_element_type=jnp.float32) + b3[...], 0.0)
    hc = jnp.concatenate([h, cos], axis=0)
    out[...] = jnp.dot(Wot[...], hc, preferred_element_type=jnp.float32) + bo[...]


def _mlp_t(xT, W1t, b1, W2t, b2, W3t, b3, Wot, bo):
    blk = 8192
    grid = (_B // blk,)
    col = lambda h: pl.BlockSpec((h, blk), lambda i: (0, i))
    rep = lambda a, b: pl.BlockSpec((a, b), lambda i: (0, 0))
    return pl.pallas_call(
        _mlp_body,
        grid=grid,
        in_specs=[
            col(128),
            rep(64, 128), rep(64, 1), rep(32, 64), rep(32, 1),
            rep(16, 32), rep(16, 1), rep(1, 17), rep(1, 1),
        ],
        out_specs=pl.BlockSpec((1, blk), lambda i: (0, i)),
        out_shape=jax.ShapeDtypeStruct((1, _B), jnp.float32),
    )(xT, W1t, b1, W2t, b2, W3t, b3, Wot, bo)


def kernel(user_idx, item_idx, U0, U1, U2, I0, I1, I2,
           W1, b1, W2, b2, W3, b3, Wo, bo):
    ui = user_idx.astype(jnp.int32)
    ii = item_idx.astype(jnp.int32)
    xT3 = _make_gather()(
        ui[:, 0], ui[:, 1], ui[:, 2], ii[:, 0], ii[:, 1], ii[:, 2],
        U0.T, U1.T, U2.T, I0.T, I1.T, I2.T)
    xT = xT3.reshape(128, _B)
    outT = _mlp_t(xT, W1.T, b1.reshape(-1, 1), W2.T, b2.reshape(-1, 1),
                  W3.T, b3.reshape(-1, 1), Wo.T, bo.reshape(1, 1))
    return outT.reshape(_B, 1)


# speedup vs baseline: 3.9507x; 1.1394x over previous
import functools

import jax
import jax.numpy as jnp
from jax import lax
from jax.experimental import pallas as pl
from jax.experimental.pallas import tpu as pltpu
from jax.experimental.pallas import tpu_sc as plsc

_B = 16384
_V = 100000
_NC = 2
_NS = 16
_NW = _NC * _NS
_HALF = _B // 2
_EPS = 1e-8


def _gather_body(u0i, u1i, u2i, i0i, i1i, i2i,
                 U0t, U1t, U2t, I0t, I1t, I2t,
                 out, trow, idxv, outv, osem):
    wid = lax.axis_index("s") * _NC + lax.axis_index("c")
    sub = wid - 16

    def stage(tab, row, idx_hbm):
        pltpu.sync_copy(tab.at[row], trow)
        pltpu.sync_copy(idx_hbm, idxv)

    def gather_to(drow):
        for half in range(2):
            @pl.loop(0, _HALF // 16)
            def _(i):
                v = idxv[pl.ds(half * _HALF + i * 16, 16)]
                outv[pl.ds(i * 16, 16)] = plsc.load_gather(trow, [v])
            pltpu.sync_copy(outv, out.at[drow, pl.ds(half * _HALF, _HALF)])

    stage(U0t, wid, u0i)
    gather_to(wid)
    @pl.when(wid < 16)
    def _():
        stage(U1t, wid, u1i)
    @pl.when(wid >= 16)
    def _():
        stage(U2t, sub, u2i)
    gather_to(32 + wid)
    stage(I0t, wid, i0i)
    gather_to(64 + wid)
    @pl.when(wid < 16)
    def _():
        stage(I1t, wid, i1i)
    @pl.when(wid >= 16)
    def _():
        stage(I2t, sub, i2i)
    gather_to(96 + wid)
    del osem


@functools.cache
def _make_gather():
    return functools.partial(
        pl.kernel,
        out_type=jax.ShapeDtypeStruct((128, _B), jnp.float32),
        mesh=plsc.VectorSubcoreMesh(core_axis_name="c", subcore_axis_name="s"),
        compiler_params=pltpu.CompilerParams(
            use_tc_tiling_on_sc=True, needs_layout_passes=False),
        scratch_types=[
            pltpu.VMEM((_V,), jnp.float32),
            pltpu.VMEM((_B,), jnp.int32),
            pltpu.VMEM((_HALF,), jnp.float32),
            pltpu.SemaphoreType.DMA,
        ],
    )(_gather_body)


def _mlp_body(xT, W1t, b1, W2t, b2, W3t, b3, Wot, bo, out):
    x = xT[...]
    a = x[33:64, :]
    s = jnp.sum(a * a, axis=0, keepdims=True)
    na = jnp.sqrt(s)
    d = jnp.maximum(na, _EPS)
    cos = s / (d * d)
    h = jnp.maximum(jnp.dot(W1t[...], x, preferred_element_type=jnp.float32) + b1[...], 0.0)
    h = jnp.maximum(jnp.dot(W2t[...], h, preferred_element_type=jnp.float32) + b2[...], 0.0)
    h = jnp.maximum(jnp.dot(W3t[...], h, preferred_element_type=jnp.float32) + b3[...], 0.0)
    hc = jnp.concatenate([h, cos], axis=0)
    out[...] = jnp.dot(Wot[...], hc, preferred_element_type=jnp.float32) + bo[...]


def _mlp_t(xT, W1t, b1, W2t, b2, W3t, b3, Wot, bo):
    blk = 8192
    grid = (_B // blk,)
    col = lambda h: pl.BlockSpec((h, blk), lambda i: (0, i))
    rep = lambda a, b: pl.BlockSpec((a, b), lambda i: (0, 0))
    return pl.pallas_call(
        _mlp_body,
        grid=grid,
        in_specs=[
            col(128),
            rep(64, 128), rep(64, 1), rep(32, 64), rep(32, 1),
            rep(16, 32), rep(16, 1), rep(1, 17), rep(1, 1),
        ],
        out_specs=pl.BlockSpec((1, blk), lambda i: (0, i)),
        out_shape=jax.ShapeDtypeStruct((1, _B), jnp.float32),
    )(xT, W1t, b1, W2t, b2, W3t, b3, Wot, bo)


def kernel(user_idx, item_idx, U0, U1, U2, I0, I1, I2,
           W1, b1, W2, b2, W3, b3, Wo, bo):
    ui = user_idx.astype(jnp.int32)
    ii = item_idx.astype(jnp.int32)
    xT = _make_gather()(
        ui[:, 0], ui[:, 1], ui[:, 2], ii[:, 0], ii[:, 1], ii[:, 2],
        U0.T, U1.T, U2.T, I0.T, I1.T, I2.T)
    outT = _mlp_t(xT, W1.T, b1.reshape(-1, 1), W2.T, b2.reshape(-1, 1),
                  W3.T, b3.reshape(-1, 1), Wo.T, bo.reshape(1, 1))
    return outT.reshape(_B, 1)
